# baseline (speedup 1.0000x reference)
import functools

import jax
import jax.numpy as jnp
from jax import lax
from jax.experimental import pallas as pl
from jax.experimental.pallas import tpu as pltpu
from jax.experimental.pallas import tpu_sc as plsc

N_CODES = 1024
DIM = 64
TOKENS = 1024


def _argmin_body(z_ref, e_ref, idx_ref):
    zb = z_ref[0]
    e = e_ref[...]
    eh = 0.5 * jnp.sum(e * e, axis=1, keepdims=True)
    scores = lax.dot_general(
        e, zb, (((1,), (0,)), ((), ())),
        preferred_element_type=jnp.float32)
    d = eh - scores
    dmin = jnp.min(d, axis=0, keepdims=True)
    iota = lax.broadcasted_iota(jnp.int32, (N_CODES, TOKENS), 0)
    masked = jnp.where(d == dmin, iota, jnp.int32(N_CODES))
    idx_ref[0, 0, :] = jnp.min(masked, axis=0)


def _tc_argmin(zf, embedding_weight):
    B = zf.shape[0]
    return pl.pallas_call(
        _argmin_body,
        grid=(B,),
        in_specs=[
            pl.BlockSpec((1, DIM, TOKENS), lambda i: (i, 0, 0)),
            pl.BlockSpec((N_CODES, DIM), lambda i: (0, 0)),
        ],
        out_specs=pl.BlockSpec((1, 1, TOKENS), lambda i: (i, 0, 0)),
        out_shape=jax.ShapeDtypeStruct((B, 1, TOKENS), jnp.int32),
    )(zf, embedding_weight)


PADDED_DIM = 128
IDX_CHUNK = 128


def _make_sc_gather(n_tokens):
    info = plsc.get_sparse_core_info()
    nc, ns = info.num_cores, info.num_subcores
    nw = nc * ns
    per_w = n_tokens // nw
    n_chunks = per_w // IDX_CHUNK
    mesh = plsc.VectorSubcoreMesh(core_axis_name="c", subcore_axis_name="s")

    @functools.partial(
        pl.kernel, mesh=mesh,
        out_type=jax.ShapeDtypeStruct((n_tokens, PADDED_DIM), jnp.float32),
        scratch_types=[
            pltpu.VMEM((n_chunks, IDX_CHUNK), jnp.int32),
            pltpu.VMEM((per_w, PADDED_DIM), jnp.float32),
            pltpu.SemaphoreType.DMA,
        ],
    )
    def gather(table_hbm, idx_hbm, out_hbm, idx_v, rows_v, sem):
        wid = lax.axis_index("s") * nc + lax.axis_index("c")
        pltpu.sync_copy(idx_hbm.at[wid], idx_v)
        copies = [
            pltpu.async_copy(
                table_hbm.at[idx_v.at[j]],
                rows_v.at[pl.ds(j * IDX_CHUNK, IDX_CHUNK)], sem)
            for j in range(n_chunks)
        ]
        for c in copies:
            c.wait()
        pltpu.sync_copy(rows_v, out_hbm.at[pl.ds(wid * per_w, per_w)])

    return gather


def kernel(z, embedding_weight):
    B, C, H, W = z.shape
    zf = z.reshape(B, C, H * W)
    idx_out = _tc_argmin(zf, embedding_weight)
    n_tokens = B * H * W
    nw = 32
    e_pad = jnp.concatenate(
        [embedding_weight,
         jnp.zeros((N_CODES, PADDED_DIM - DIM), jnp.float32)], axis=1)
    gathered = _make_sc_gather(n_tokens)(
        e_pad, idx_out.reshape(nw, -1, IDX_CHUNK))
    zq = (gathered[:, :DIM].reshape(B, H * W, C)
          .transpose(0, 2, 1).reshape(B, C, H, W))
    return idx_out.reshape(B, 1, H, W), zq
